# baseline (speedup 1.0000x reference)
import jax
import jax.numpy as jnp
from jax.experimental import pallas as pl
from jax.experimental.pallas import tpu as pltpu


def _concat_body(pos_ref, h_ref, emb_ref, out_ref):
    s = h_ref.shape[1]
    out_ref[0, pl.ds(1, s), :] = h_ref[0]
    out_ref[0, pl.ds(0, 1), :] = emb_ref[pl.ds(pos_ref[0], 1), :]


def kernel(hidden_states, position, embeddings):
    b, s, d = hidden_states.shape
    depth = embeddings.shape[0]
    pos_arr = jnp.asarray(position, jnp.int32).reshape((1,))
    return pl.pallas_call(
        _concat_body,
        grid=(b,),
        out_shape=jax.ShapeDtypeStruct((b, s + 1, d), hidden_states.dtype),
        in_specs=[
            pl.BlockSpec(memory_space=pltpu.SMEM),
            pl.BlockSpec((1, s, d), lambda i: (i, 0, 0)),
            pl.BlockSpec((depth, d), lambda i: (0, 0)),
        ],
        out_specs=pl.BlockSpec((1, s + 1, d), lambda i: (i, 0, 0)),
    )(pos_arr, hidden_states, embeddings)
